# Initial kernel scaffold; baseline (speedup 1.0000x reference)
import jax
import jax.numpy as jnp
from jax.experimental import pallas as pl


def kernel(x, k_input, k_process, q1_w, q1_b, ln_w, ln_b, q2_w, q2_b, neuron_keys, input_patterns, process_weights, process_outputs):
    raise NotImplementedError("write your pallas kernel here")


# speedup vs baseline: 5.0627x; 5.0627x over previous
import functools
import math

import jax
import jax.numpy as jnp
from jax.experimental import pallas as pl
from jax.experimental.pallas import tpu as pltpu


def _gelu_exact(v):
    return 0.5 * v * (1.0 + jax.lax.erf(v / jnp.float32(math.sqrt(2.0))))


def _router_body(x_ref, q1w_ref, q1b_ref, lnw_ref, lnb_ref, q2w_ref, q2b_ref,
                 nk_ref, mask_ref, *, k_sel):
    gmax = jnp.max(x_ref[0], axis=0, keepdims=True)
    h = jax.lax.dot_general(gmax, q1w_ref[...], (((1,), (1,)), ((), ())),
                            preferred_element_type=jnp.float32) + q1b_ref[...]
    h = _gelu_exact(h)
    mu = jnp.mean(h, axis=-1, keepdims=True)
    var = jnp.mean((h - mu) ** 2, axis=-1, keepdims=True)
    h = (h - mu) / jnp.sqrt(var + 1e-5) * lnw_ref[...] + lnb_ref[...]
    q = jax.lax.dot_general(h, q2w_ref[...], (((1,), (1,)), ((), ())),
                            preferred_element_type=jnp.float32) + q2b_ref[...]
    d_routing = q.shape[-1]
    logits = jax.lax.dot_general(q, nk_ref[...], (((1,), (1,)), ((), ())),
                                 preferred_element_type=jnp.float32)
    logits = logits / jnp.float32(math.sqrt(d_routing))

    n_in = logits.shape[-1]
    u = jax.lax.bitcast_convert_type(logits, jnp.uint32)
    msb = jnp.uint32(0x80000000)
    ukey = jnp.where(u >= msb, ~u, u | msb)

    def tbody(i, t):
        cand = t | (jnp.uint32(1) << jnp.uint32(31 - i))
        cnt = jnp.sum((ukey >= cand).astype(jnp.int32))
        return jnp.where(cnt >= k_sel, cand, t)

    t = jax.lax.fori_loop(0, 32, tbody, jnp.uint32(0))

    c_gt = jnp.sum((ukey > t).astype(jnp.int32))
    need = k_sel - c_gt
    idx = jax.lax.broadcasted_iota(jnp.int32, (1, n_in), 1)
    eq = ukey == t

    nbits = max(1, (n_in - 1).bit_length())

    def jbody(i, lh):
        lo, hi = lh
        mid = (lo + hi) // 2
        g = jnp.sum((eq & (idx <= mid)).astype(jnp.int32))
        pred = g >= need
        return jnp.where(pred, lo, mid + 1), jnp.where(pred, mid, hi)

    lo, _ = jax.lax.fori_loop(0, nbits, jbody,
                              (jnp.int32(0), jnp.int32(n_in - 1)))
    sel = (ukey > t) | (eq & (idx <= lo) & (need > 0))
    mask_ref[0] = sel.astype(jnp.float32)


def _ffn_body(x_ref, mask_ref, ip_ref, pw_ref, po_ref, out_ref, acc_ref):
    k = pl.program_id(2)
    nk = pl.num_programs(2)
    acts = jax.lax.dot_general(x_ref[0], ip_ref[...], (((1,), (1,)), ((), ())),
                               preferred_element_type=jnp.float32)
    acts = _gelu_exact(acts) * mask_ref[0]
    contrib = jax.lax.dot_general(acts, pw_ref[...], (((1,), (1,)), ((), ())),
                                  preferred_element_type=jnp.float32)

    @pl.when(k == 0)
    def _():
        acc_ref[...] = contrib

    @pl.when(k > 0)
    def _():
        acc_ref[...] += contrib

    @pl.when(k == nk - 1)
    def _():
        pacts = _gelu_exact(acc_ref[...])
        out_ref[0] = jax.lax.dot_general(pacts, po_ref[...],
                                         (((1,), (0,)), ((), ())),
                                         preferred_element_type=jnp.float32)


def kernel(x, k_input, k_process, q1_w, q1_b, ln_w, ln_b, q2_w, q2_b,
           neuron_keys, input_patterns, process_weights, process_outputs):
    B, S, D = x.shape
    N_IN, D_R = neuron_keys.shape
    N_PR = process_weights.shape[0]
    K_SEL = N_PR

    mask = pl.pallas_call(
        functools.partial(_router_body, k_sel=K_SEL),
        grid=(B,),
        in_specs=[
            pl.BlockSpec((1, S, D), lambda b: (b, 0, 0)),
            pl.BlockSpec(q1_w.shape, lambda b: (0, 0)),
            pl.BlockSpec((1, q1_b.shape[0]), lambda b: (0, 0)),
            pl.BlockSpec((1, ln_w.shape[0]), lambda b: (0, 0)),
            pl.BlockSpec((1, ln_b.shape[0]), lambda b: (0, 0)),
            pl.BlockSpec(q2_w.shape, lambda b: (0, 0)),
            pl.BlockSpec((1, q2_b.shape[0]), lambda b: (0, 0)),
            pl.BlockSpec(neuron_keys.shape, lambda b: (0, 0)),
        ],
        out_specs=pl.BlockSpec((1, 1, N_IN), lambda b: (b, 0, 0)),
        out_shape=jax.ShapeDtypeStruct((B, 1, N_IN), jnp.float32),
    )(x, q1_w, q1_b.reshape(1, -1), ln_w.reshape(1, -1), ln_b.reshape(1, -1),
      q2_w, q2_b.reshape(1, -1), neuron_keys)

    TS = min(512, S)
    KT = min(1024, N_IN)
    out = pl.pallas_call(
        _ffn_body,
        grid=(B, S // TS, N_IN // KT),
        in_specs=[
            pl.BlockSpec((1, TS, D), lambda b, s, k: (b, s, 0)),
            pl.BlockSpec((1, 1, KT), lambda b, s, k: (b, 0, k)),
            pl.BlockSpec((KT, D), lambda b, s, k: (k, 0)),
            pl.BlockSpec((N_PR, KT), lambda b, s, k: (0, k)),
            pl.BlockSpec((N_PR, D), lambda b, s, k: (0, 0)),
        ],
        out_specs=pl.BlockSpec((1, TS, D), lambda b, s, k: (b, s, 0)),
        out_shape=jax.ShapeDtypeStruct((B, S, D), jnp.float32),
        scratch_shapes=[pltpu.VMEM((TS, N_PR), jnp.float32)],
        compiler_params=pltpu.CompilerParams(
            dimension_semantics=("parallel", "arbitrary", "arbitrary"),
        ),
    )(x, mask, input_patterns, process_weights, process_outputs)
    return out


# speedup vs baseline: 5.1401x; 1.0153x over previous
import functools
import math

import jax
import jax.numpy as jnp
from jax.experimental import pallas as pl
from jax.experimental.pallas import tpu as pltpu


def _gelu_exact(v):
    return 0.5 * v * (1.0 + jax.lax.erf(v / jnp.float32(math.sqrt(2.0))))


def _router_body(x_ref, q1w_ref, q1b_ref, lnw_ref, lnb_ref, q2w_ref, q2b_ref,
                 nk_ref, mask_ref, *, k_sel):
    gmax = jnp.max(x_ref[0], axis=0, keepdims=True)
    h = jax.lax.dot_general(gmax, q1w_ref[...], (((1,), (1,)), ((), ())),
                            preferred_element_type=jnp.float32) + q1b_ref[...]
    h = _gelu_exact(h)
    mu = jnp.mean(h, axis=-1, keepdims=True)
    var = jnp.mean((h - mu) ** 2, axis=-1, keepdims=True)
    h = (h - mu) / jnp.sqrt(var + 1e-5) * lnw_ref[...] + lnb_ref[...]
    q = jax.lax.dot_general(h, q2w_ref[...], (((1,), (1,)), ((), ())),
                            preferred_element_type=jnp.float32) + q2b_ref[...]
    d_routing = q.shape[-1]
    logits = jax.lax.dot_general(q, nk_ref[...], (((1,), (1,)), ((), ())),
                                 preferred_element_type=jnp.float32)
    logits = logits / jnp.float32(math.sqrt(d_routing))

    n_in = logits.shape[-1]
    u = jax.lax.bitcast_convert_type(logits, jnp.uint32)
    msb = jnp.uint32(0x80000000)
    ukey = jnp.where(u >= msb, ~u, u | msb)

    def tbody(i, t):
        cand = t | (jnp.uint32(1) << jnp.uint32(31 - i))
        cnt = jnp.sum((ukey >= cand).astype(jnp.int32))
        return jnp.where(cnt >= k_sel, cand, t)

    t = jax.lax.fori_loop(0, 32, tbody, jnp.uint32(0))

    c_gt = jnp.sum((ukey > t).astype(jnp.int32))
    need = k_sel - c_gt
    idx = jax.lax.broadcasted_iota(jnp.int32, (1, n_in), 1)
    eq = ukey == t

    nbits = max(1, (n_in - 1).bit_length())

    def jbody(i, lh):
        lo, hi = lh
        mid = (lo + hi) // 2
        g = jnp.sum((eq & (idx <= mid)).astype(jnp.int32))
        pred = g >= need
        return jnp.where(pred, lo, mid + 1), jnp.where(pred, mid, hi)

    lo, _ = jax.lax.fori_loop(0, nbits, jbody,
                              (jnp.int32(0), jnp.int32(n_in - 1)))
    sel = (ukey > t) | (eq & (idx <= lo) & (need > 0))
    mask_ref[0] = sel.astype(jnp.float32)


def _ffn_body_bf16(x_ref, mask_ref, ip_ref, pw_ref, po_ref, out_ref):
    acts = jax.lax.dot_general(x_ref[0], ip_ref[...], (((1,), (1,)), ((), ())),
                               preferred_element_type=jnp.float32)
    am = (_gelu_exact(acts) * mask_ref[0]).astype(jnp.bfloat16)
    pacts = jax.lax.dot_general(am, pw_ref[...], (((1,), (1,)), ((), ())),
                                preferred_element_type=jnp.float32)
    pacts = _gelu_exact(pacts).astype(jnp.bfloat16)
    out_ref[0] = jax.lax.dot_general(pacts, po_ref[...],
                                     (((1,), (0,)), ((), ())),
                                     preferred_element_type=jnp.float32)


def _ffn_body(x_ref, mask_ref, ip_ref, pw_ref, po_ref, out_ref, acc_ref):
    k = pl.program_id(2)
    nk = pl.num_programs(2)
    acts = jax.lax.dot_general(x_ref[0], ip_ref[...], (((1,), (1,)), ((), ())),
                               preferred_element_type=jnp.float32)
    acts = _gelu_exact(acts) * mask_ref[0]
    contrib = jax.lax.dot_general(acts, pw_ref[...], (((1,), (1,)), ((), ())),
                                  preferred_element_type=jnp.float32)

    @pl.when(k == 0)
    def _():
        acc_ref[...] = contrib

    @pl.when(k > 0)
    def _():
        acc_ref[...] += contrib

    @pl.when(k == nk - 1)
    def _():
        pacts = _gelu_exact(acc_ref[...])
        out_ref[0] = jax.lax.dot_general(pacts, po_ref[...],
                                         (((1,), (0,)), ((), ())),
                                         preferred_element_type=jnp.float32)


def kernel(x, k_input, k_process, q1_w, q1_b, ln_w, ln_b, q2_w, q2_b,
           neuron_keys, input_patterns, process_weights, process_outputs):
    B, S, D = x.shape
    N_IN, D_R = neuron_keys.shape
    N_PR = process_weights.shape[0]
    K_SEL = N_PR

    mask = pl.pallas_call(
        functools.partial(_router_body, k_sel=K_SEL),
        grid=(B,),
        in_specs=[
            pl.BlockSpec((1, S, D), lambda b: (b, 0, 0)),
            pl.BlockSpec(q1_w.shape, lambda b: (0, 0)),
            pl.BlockSpec((1, q1_b.shape[0]), lambda b: (0, 0)),
            pl.BlockSpec((1, ln_w.shape[0]), lambda b: (0, 0)),
            pl.BlockSpec((1, ln_b.shape[0]), lambda b: (0, 0)),
            pl.BlockSpec(q2_w.shape, lambda b: (0, 0)),
            pl.BlockSpec((1, q2_b.shape[0]), lambda b: (0, 0)),
            pl.BlockSpec(neuron_keys.shape, lambda b: (0, 0)),
        ],
        out_specs=pl.BlockSpec((1, 1, N_IN), lambda b: (b, 0, 0)),
        out_shape=jax.ShapeDtypeStruct((B, 1, N_IN), jnp.float32),
    )(x, q1_w, q1_b.reshape(1, -1), ln_w.reshape(1, -1), ln_b.reshape(1, -1),
      q2_w, q2_b.reshape(1, -1), neuron_keys)

    TS = min(256, S)
    out = pl.pallas_call(
        _ffn_body_bf16,
        grid=(B, S // TS),
        in_specs=[
            pl.BlockSpec((1, TS, D), lambda b, s: (b, s, 0)),
            pl.BlockSpec((1, 1, N_IN), lambda b, s: (b, 0, 0)),
            pl.BlockSpec((N_IN, D), lambda b, s: (0, 0)),
            pl.BlockSpec((N_PR, N_IN), lambda b, s: (0, 0)),
            pl.BlockSpec((N_PR, D), lambda b, s: (0, 0)),
        ],
        out_specs=pl.BlockSpec((1, TS, D), lambda b, s: (b, s, 0)),
        out_shape=jax.ShapeDtypeStruct((B, S, D), jnp.float32),
        compiler_params=pltpu.CompilerParams(
            dimension_semantics=("parallel", "arbitrary"),
        ),
    )(x.astype(jnp.bfloat16), mask, input_patterns.astype(jnp.bfloat16),
      process_weights.astype(jnp.bfloat16), process_outputs.astype(jnp.bfloat16))
    return out
